# baseline (speedup 1.0000x reference)
import functools

import jax
import jax.numpy as jnp
from jax import lax
from jax.experimental import pallas as pl
from jax.experimental.pallas import tpu as pltpu
from jax.experimental.pallas import tpu_sc as plsc

NC = 2
NS = 16
NW = NC * NS

XB = 16
NBUF = 4


def _make_sc_kernel(B, S, D, dtype):
    rows_per_w = S // NW
    nsub = rows_per_w // XB
    nblocks = nsub * B
    mesh = plsc.VectorSubcoreMesh(core_axis_name="c", subcore_axis_name="s")

    @functools.partial(
        pl.kernel,
        mesh=mesh,
        out_type=jax.ShapeDtypeStruct((B, S, D), dtype),
        scratch_types=[
            pltpu.VMEM((2 * XB, D), dtype),
            pltpu.VMEM((XB, D), dtype),
            pltpu.VMEM((XB, D), dtype),
            pltpu.VMEM((XB, D), dtype),
            pltpu.VMEM((XB, D), dtype),
            pltpu.SemaphoreType.DMA,
            pltpu.SemaphoreType.DMA,
            pltpu.SemaphoreType.DMA,
            pltpu.SemaphoreType.DMA,
            pltpu.SemaphoreType.DMA,
            pltpu.SemaphoreType.DMA,
            pltpu.SemaphoreType.DMA,
            pltpu.SemaphoreType.DMA,
            pltpu.SemaphoreType.DMA,
        ],
    )
    def k(x_hbm, pos_hbm, out_hbm, pos_buf, xb0, xb1, xb2, xb3,
          psem, ls0, ls1, ls2, ls3, ss0, ss1, ss2, ss3):
        xbufs = (xb0, xb1, xb2, xb3)
        lsems = (ls0, ls1, ls2, ls3)
        ssems = (ss0, ss1, ss2, ss3)
        wid = lax.axis_index("s") * NC + lax.axis_index("c")
        s0 = wid * rows_per_w

        def compute_block(xbuf, h):
            @plsc.parallel_loop(0, XB, step=1, unroll=4)
            def row_body(r):
                pr = h + r
                for u in range(D // 16):
                    v = pos_buf[pr, pl.ds(u * 16, 16)]
                    plsc.addupdate(xbuf.at[r, pl.ds(u * 16, 16)], v)

        def wait_pos():
            pltpu.make_async_copy(
                pos_hbm.at[pl.ds(0, XB), :],
                pos_buf.at[pl.ds(0, XB), :], psem,
            ).wait()

        pltpu.async_copy(
            pos_hbm.at[pl.ds(s0, XB), :], pos_buf.at[pl.ds(0, XB), :], psem
        )
        for g in range(3):
            pltpu.async_copy(
                x_hbm.at[g, pl.ds(s0, XB), :], xbufs[g], lsems[g]
            )

        def sub_body(sub, _):
            h = (sub & 1) * XB
            rows = s0 + sub * XB
            wait_pos()

            def pos_prefetch():
                nxt = sub + 1
                pltpu.async_copy(
                    pos_hbm.at[pl.ds(s0 + nxt * XB, XB), :],
                    pos_buf.at[pl.ds((nxt & 1) * XB, XB), :], psem,
                )

            pl.when(sub + 1 < nsub)(pos_prefetch)

            for p in range(B):
                g = sub * B + p
                q = (p + 3) % NBUF

                pltpu.make_async_copy(
                    x_hbm.at[0, pl.ds(0, XB), :], xbufs[p], lsems[p]
                ).wait()

                def prefetch(g=g, q=q):
                    g2 = g + 3
                    b2 = g2 & (B - 1)
                    rows2 = s0 + (g2 >> 2) * XB

                    def wait_store(q=q):
                        pltpu.make_async_copy(
                            xbufs[q], out_hbm.at[0, pl.ds(0, XB), :], ssems[q]
                        ).wait()

                    pl.when(g >= 1)(wait_store)
                    pltpu.async_copy(
                        x_hbm.at[b2, pl.ds(rows2, XB), :], xbufs[q], lsems[q]
                    )

                pl.when(g + 3 < nblocks)(prefetch)

                compute_block(xbufs[p], h)
                pltpu.async_copy(
                    xbufs[p], out_hbm.at[p, pl.ds(rows, XB), :], ssems[p]
                )
            return 0

        lax.fori_loop(0, nsub, sub_body, 0)

        for p in range(NBUF):
            pltpu.make_async_copy(
                xbufs[p], out_hbm.at[0, pl.ds(0, XB), :], ssems[p]
            ).wait()

    return k


def kernel(x, pos_table):
    B, S, D = x.shape
    pos = pos_table[:S]
    return _make_sc_kernel(B, S, D, x.dtype)(x, pos)


# speedup vs baseline: 1.2263x; 1.2263x over previous
import functools

import jax
import jax.numpy as jnp
from jax import lax
from jax.experimental import pallas as pl
from jax.experimental.pallas import tpu as pltpu
from jax.experimental.pallas import tpu_sc as plsc

NC = 2
NS = 16
NW = NC * NS

XB = 16
NBUF = 4


def _make_sc_kernel(B, S, D, dtype):
    rows_per_w = S // NW
    nsub = rows_per_w // XB
    nblocks = nsub * B
    mesh = plsc.VectorSubcoreMesh(core_axis_name="c", subcore_axis_name="s")

    @functools.partial(
        pl.kernel,
        mesh=mesh,
        out_type=jax.ShapeDtypeStruct((B, S, D), dtype),
        scratch_types=[
            pltpu.VMEM((2 * XB, D), dtype),
            pltpu.VMEM((XB, D), dtype),
            pltpu.VMEM((XB, D), dtype),
            pltpu.VMEM((XB, D), dtype),
            pltpu.VMEM((XB, D), dtype),
            pltpu.SemaphoreType.DMA,
            pltpu.SemaphoreType.DMA,
            pltpu.SemaphoreType.DMA,
            pltpu.SemaphoreType.DMA,
            pltpu.SemaphoreType.DMA,
            pltpu.SemaphoreType.DMA,
            pltpu.SemaphoreType.DMA,
            pltpu.SemaphoreType.DMA,
            pltpu.SemaphoreType.DMA,
        ],
    )
    def k(x_hbm, pos_hbm, out_hbm, pos_buf, xb0, xb1, xb2, xb3,
          psem, ls0, ls1, ls2, ls3, ss0, ss1, ss2, ss3):
        xbufs = (xb0, xb1, xb2, xb3)
        lsems = (ls0, ls1, ls2, ls3)
        ssems = (ss0, ss1, ss2, ss3)
        wid = lax.axis_index("s") * NC + lax.axis_index("c")
        s0 = wid * rows_per_w

        def compute_block(xbuf, h):
            @plsc.parallel_loop(0, XB, step=1, unroll=4)
            def row_body(r):
                pr = h + r
                for u in range(D // 16):
                    v = pos_buf[pr, pl.ds(u * 16, 16)]
                    plsc.addupdate(xbuf.at[r, pl.ds(u * 16, 16)], v)

        def wait_pos():
            pltpu.make_async_copy(
                pos_hbm.at[pl.ds(0, XB), :],
                pos_buf.at[pl.ds(0, XB), :], psem,
            ).wait()

        pltpu.async_copy(
            pos_hbm.at[pl.ds(s0, XB), :], pos_buf.at[pl.ds(0, XB), :], psem
        )
        for g in range(2):
            pltpu.async_copy(
                x_hbm.at[g, pl.ds(s0, XB), :], xbufs[g], lsems[g]
            )

        def sub_body(sub, _):
            h = (sub & 1) * XB
            rows = s0 + sub * XB
            wait_pos()

            def pos_prefetch():
                nxt = sub + 1
                pltpu.async_copy(
                    pos_hbm.at[pl.ds(s0 + nxt * XB, XB), :],
                    pos_buf.at[pl.ds((nxt & 1) * XB, XB), :], psem,
                )

            pl.when(sub + 1 < nsub)(pos_prefetch)

            for p in range(B):
                g = sub * B + p
                q = (p + 2) % NBUF

                pltpu.make_async_copy(
                    x_hbm.at[0, pl.ds(0, XB), :], xbufs[p], lsems[p]
                ).wait()

                def prefetch(g=g, q=q):
                    g2 = g + 2
                    b2 = g2 & (B - 1)
                    rows2 = s0 + (g2 >> 2) * XB

                    def wait_store(q=q):
                        pltpu.make_async_copy(
                            xbufs[q], out_hbm.at[0, pl.ds(0, XB), :], ssems[q]
                        ).wait()

                    pl.when(g >= 2)(wait_store)
                    pltpu.async_copy(
                        x_hbm.at[b2, pl.ds(rows2, XB), :], xbufs[q], lsems[q]
                    )

                pl.when(g + 2 < nblocks)(prefetch)

                compute_block(xbufs[p], h)
                pltpu.async_copy(
                    xbufs[p], out_hbm.at[p, pl.ds(rows, XB), :], ssems[p]
                )
            return 0

        lax.fori_loop(0, nsub, sub_body, 0)

        for p in range(NBUF):
            pltpu.make_async_copy(
                xbufs[p], out_hbm.at[0, pl.ds(0, XB), :], ssems[p]
            ).wait()

    return k


def kernel(x, pos_table):
    B, S, D = x.shape
    pos = pos_table[:S]
    return _make_sc_kernel(B, S, D, x.dtype)(x, pos)
